# Initial kernel scaffold; baseline (speedup 1.0000x reference)
import functools

import jax
import jax.numpy as jnp
from jax import lax
from jax.experimental import pallas as pl
from jax.experimental.pallas import tpu as pltpu
from jax.experimental.pallas import tpu_sc as plsc

N_NODES = 10000
N_EDGES = 320000
D_IN = 128
D_HID = 128
D_OUT = 16

NC = 2
NS = 16
NW = NC * NS
E_PER_TILE = N_EDGES // NW
K = 80
G = E_PER_TILE // K
ROWS_PER_TILE = N_NODES // NS

_mesh = plsc.VectorSubcoreMesh(core_axis_name="c", subcore_axis_name="s")


@functools.partial(
    pl.kernel,
    out_type=jax.ShapeDtypeStruct((NC, N_NODES, 16), jnp.float32),
    mesh=_mesh,
    scratch_types=[
        pltpu.VMEM((G, K), jnp.int32),
        pltpu.VMEM((K, 16), jnp.float32),
        pltpu.VMEM_SHARED((N_NODES, 16), jnp.float32),
    ],
)
def _sc_degree(dst_hbm, zeros_hbm, out_hbm, dst_v, ones_v, acc):
    c = lax.axis_index("c")
    s = lax.axis_index("s")
    wid = s * NC + c
    pltpu.sync_copy(dst_hbm.at[wid], dst_v)

    def fill(i, carry):
        ones_v[i, :] = jnp.full((16,), 1.0, jnp.float32)
        return carry

    lax.fori_loop(0, K, fill, 0)
    pltpu.sync_copy(
        zeros_hbm.at[pl.ds(s * ROWS_PER_TILE, ROWS_PER_TILE)],
        acc.at[pl.ds(s * ROWS_PER_TILE, ROWS_PER_TILE)],
    )
    plsc.subcore_barrier()

    def body(g, carry):
        pltpu.sync_copy(ones_v, acc.at[dst_v.at[g]], add=True)
        return carry

    lax.fori_loop(0, G, body, 0)
    plsc.subcore_barrier()
    pltpu.sync_copy(
        acc.at[pl.ds(s * ROWS_PER_TILE, ROWS_PER_TILE)],
        out_hbm.at[c, pl.ds(s * ROWS_PER_TILE, ROWS_PER_TILE)],
    )


def _make_sc_conv(width):
    @functools.partial(
        pl.kernel,
        out_type=jax.ShapeDtypeStruct((NC, N_NODES, width), jnp.float32),
        mesh=_mesh,
        scratch_types=[
            pltpu.VMEM((G, K), jnp.int32),
            pltpu.VMEM((G, K), jnp.int32),
            pltpu.VMEM((K, width), jnp.float32),
            pltpu.VMEM_SHARED((N_NODES, width), jnp.float32),
            pltpu.SemaphoreType.DMA,
        ],
    )
    def conv(src_hbm, dst_hbm, z_hbm, zeros_hbm, out_hbm,
             src_v, dst_v, gbuf, acc, sem):
        c = lax.axis_index("c")
        s = lax.axis_index("s")
        wid = s * NC + c
        pltpu.sync_copy(src_hbm.at[wid], src_v)
        pltpu.sync_copy(dst_hbm.at[wid], dst_v)
        pltpu.sync_copy(
            zeros_hbm.at[pl.ds(s * ROWS_PER_TILE, ROWS_PER_TILE)],
            acc.at[pl.ds(s * ROWS_PER_TILE, ROWS_PER_TILE)],
        )
        plsc.subcore_barrier()

        def body(g, carry):
            pltpu.async_copy(z_hbm.at[src_v.at[g]], gbuf, sem).wait()
            pltpu.sync_copy(gbuf, acc.at[dst_v.at[g]], add=True)
            return carry

        lax.fori_loop(0, G, body, 0)
        plsc.subcore_barrier()
        pltpu.sync_copy(
            acc.at[pl.ds(s * ROWS_PER_TILE, ROWS_PER_TILE)],
            out_hbm.at[c, pl.ds(s * ROWS_PER_TILE, ROWS_PER_TILE)],
        )

    return conv


_sc_conv128 = _make_sc_conv(D_HID)
_sc_conv16 = _make_sc_conv(D_OUT)


BLK = 1000
GRID = (N_NODES // BLK,)


def _dis_from(degp_ref):
    deg = degp_ref[0, :, 0:1] + degp_ref[1, :, 0:1]
    return lax.rsqrt(deg)


def _lin1_body(x_ref, w_ref, degp_ref, o_ref):
    y = jnp.dot(x_ref[...], w_ref[...], preferred_element_type=jnp.float32)
    o_ref[...] = y * _dis_from(degp_ref)


def _tc_lin1(feature, W1, degp):
    return pl.pallas_call(
        _lin1_body,
        grid=GRID,
        in_specs=[
            pl.BlockSpec((BLK, D_IN), lambda i: (i, 0)),
            pl.BlockSpec((D_IN, D_HID), lambda i: (0, 0)),
            pl.BlockSpec((NC, BLK, 16), lambda i: (0, i, 0)),
        ],
        out_specs=pl.BlockSpec((BLK, D_HID), lambda i: (i, 0)),
        out_shape=jax.ShapeDtypeStruct((N_NODES, D_HID), jnp.float32),
    )(feature, W1, degp)


def _mid_body(z1_ref, p_ref, degp_ref, b1_ref, w2_ref, o_ref):
    dis = _dis_from(degp_ref)
    accv = z1_ref[...] + p_ref[0] + p_ref[1]
    h = jnp.maximum(accv * dis + b1_ref[...], 0.0)
    y2 = jnp.dot(h, w2_ref[...], preferred_element_type=jnp.float32)
    o_ref[...] = y2 * dis


def _tc_mid(z1, p, degp, b1, W2):
    return pl.pallas_call(
        _mid_body,
        grid=GRID,
        in_specs=[
            pl.BlockSpec((BLK, D_HID), lambda i: (i, 0)),
            pl.BlockSpec((NC, BLK, D_HID), lambda i: (0, i, 0)),
            pl.BlockSpec((NC, BLK, 16), lambda i: (0, i, 0)),
            pl.BlockSpec((1, D_HID), lambda i: (0, 0)),
            pl.BlockSpec((D_HID, D_OUT), lambda i: (0, 0)),
        ],
        out_specs=pl.BlockSpec((BLK, D_OUT), lambda i: (i, 0)),
        out_shape=jax.ShapeDtypeStruct((N_NODES, D_OUT), jnp.float32),
    )(z1, p, degp, b1, W2)


def _final_body(z2_ref, q_ref, degp_ref, b2_ref, o_ref):
    dis = _dis_from(degp_ref)
    accv = z2_ref[...] + q_ref[0] + q_ref[1]
    o_ref[...] = accv * dis + b2_ref[...]


def _tc_final(z2, q, degp, b2):
    return pl.pallas_call(
        _final_body,
        grid=GRID,
        in_specs=[
            pl.BlockSpec((BLK, D_OUT), lambda i: (i, 0)),
            pl.BlockSpec((NC, BLK, D_OUT), lambda i: (0, i, 0)),
            pl.BlockSpec((NC, BLK, 16), lambda i: (0, i, 0)),
            pl.BlockSpec((1, D_OUT), lambda i: (0, 0)),
        ],
        out_specs=pl.BlockSpec((BLK, D_OUT), lambda i: (i, 0)),
        out_shape=jax.ShapeDtypeStruct((N_NODES, D_OUT), jnp.float32),
    )(z2, q, degp, b2)


def kernel(edge, feature, W1, b1, W2, b2):
    edge = edge.astype(jnp.int32)
    src3 = edge[0].reshape(NW, G, K)
    dst3 = edge[1].reshape(NW, G, K)
    zeros128 = jnp.zeros((N_NODES, D_HID), jnp.float32)
    zeros16 = jnp.zeros((N_NODES, 16), jnp.float32)

    degp = _sc_degree(dst3, zeros16)
    z1 = _tc_lin1(feature, W1, degp)
    p = _sc_conv128(src3, dst3, z1, zeros128)
    z2 = _tc_mid(z1, p, degp, b1.reshape(1, -1), W2)
    q = _sc_conv16(src3, dst3, z2, zeros16)
    return _tc_final(z2, q, degp, b2.reshape(1, -1))


# speedup vs baseline: 22.9168x; 22.9168x over previous
import functools

import jax
import jax.numpy as jnp
from jax import lax
from jax.experimental import pallas as pl
from jax.experimental.pallas import tpu as pltpu
from jax.experimental.pallas import tpu_sc as plsc

N_NODES = 10000
N_EDGES = 320000
D_IN = 128
D_HID = 128
D_OUT = 16

NC = 2
NS = 16
NW = NC * NS
E_PER_TILE = N_EDGES // NW
K = 80
G = E_PER_TILE // K
ROWS_PER_TILE = N_NODES // NS

_mesh = plsc.VectorSubcoreMesh(core_axis_name="c", subcore_axis_name="s")


@functools.partial(
    pl.kernel,
    out_type=jax.ShapeDtypeStruct((NC, N_NODES, 16), jnp.float32),
    mesh=_mesh,
    compiler_params=pltpu.CompilerParams(use_tc_tiling_on_sc=False),
    scratch_types=[
        pltpu.VMEM((E_PER_TILE,), jnp.int32),
        pltpu.VMEM((K, 16), jnp.float32),
        pltpu.VMEM_SHARED((N_NODES, 16), jnp.float32),
    ],
)
def _sc_degree(dst_hbm, zeros_hbm, out_hbm, dst_v, ones_v, acc):
    c = lax.axis_index("c")
    s = lax.axis_index("s")
    wid = s * NC + c
    pltpu.sync_copy(dst_hbm.at[pl.ds(wid * E_PER_TILE, E_PER_TILE)], dst_v)

    def fill(i, carry):
        ones_v[i, :] = jnp.full((16,), 1.0, jnp.float32)
        return carry

    lax.fori_loop(0, K, fill, 0)
    pltpu.sync_copy(
        zeros_hbm.at[pl.ds(s * ROWS_PER_TILE, ROWS_PER_TILE)],
        acc.at[pl.ds(s * ROWS_PER_TILE, ROWS_PER_TILE)],
    )
    plsc.subcore_barrier()

    def body(g, carry):
        pltpu.sync_copy(ones_v, acc.at[dst_v.at[pl.ds(g * K, K)]], add=True)
        return carry

    lax.fori_loop(0, G, body, 0)
    plsc.subcore_barrier()
    pltpu.sync_copy(
        acc.at[pl.ds(s * ROWS_PER_TILE, ROWS_PER_TILE)],
        out_hbm.at[c, pl.ds(s * ROWS_PER_TILE, ROWS_PER_TILE)],
    )


def _make_sc_conv(width):
    @functools.partial(
        pl.kernel,
        out_type=jax.ShapeDtypeStruct((NC, N_NODES, width), jnp.float32),
        mesh=_mesh,
        compiler_params=pltpu.CompilerParams(use_tc_tiling_on_sc=False),
        scratch_types=[
            pltpu.VMEM((E_PER_TILE,), jnp.int32),
            pltpu.VMEM((E_PER_TILE,), jnp.int32),
            pltpu.VMEM((K, width), jnp.float32),
            pltpu.VMEM_SHARED((N_NODES, width), jnp.float32),
            pltpu.SemaphoreType.DMA,
        ],
    )
    def conv(src_hbm, dst_hbm, z_hbm, zeros_hbm, out_hbm,
             src_v, dst_v, gbuf, acc, sem):
        c = lax.axis_index("c")
        s = lax.axis_index("s")
        wid = s * NC + c
        pltpu.sync_copy(src_hbm.at[pl.ds(wid * E_PER_TILE, E_PER_TILE)], src_v)
        pltpu.sync_copy(dst_hbm.at[pl.ds(wid * E_PER_TILE, E_PER_TILE)], dst_v)
        pltpu.sync_copy(
            zeros_hbm.at[pl.ds(s * ROWS_PER_TILE, ROWS_PER_TILE)],
            acc.at[pl.ds(s * ROWS_PER_TILE, ROWS_PER_TILE)],
        )
        plsc.subcore_barrier()

        def body(g, carry):
            pltpu.async_copy(
                z_hbm.at[src_v.at[pl.ds(g * K, K)]], gbuf, sem).wait()
            pltpu.sync_copy(gbuf, acc.at[dst_v.at[pl.ds(g * K, K)]], add=True)
            return carry

        lax.fori_loop(0, G, body, 0)
        plsc.subcore_barrier()
        pltpu.sync_copy(
            acc.at[pl.ds(s * ROWS_PER_TILE, ROWS_PER_TILE)],
            out_hbm.at[c, pl.ds(s * ROWS_PER_TILE, ROWS_PER_TILE)],
        )

    return conv


_sc_conv128 = _make_sc_conv(D_HID)
_sc_conv16 = _make_sc_conv(D_OUT)


BLK = 1000
GRID = (N_NODES // BLK,)


def _dis_from(degp_ref):
    deg = degp_ref[0, :, 0:1] + degp_ref[1, :, 0:1] + 1.0
    return lax.rsqrt(deg)


def _lin1_body(x_ref, w_ref, degp_ref, o_ref):
    y = jnp.dot(x_ref[...], w_ref[...], preferred_element_type=jnp.float32)
    o_ref[...] = y * _dis_from(degp_ref)


def _tc_lin1(feature, W1, degp):
    return pl.pallas_call(
        _lin1_body,
        grid=GRID,
        in_specs=[
            pl.BlockSpec((BLK, D_IN), lambda i: (i, 0)),
            pl.BlockSpec((D_IN, D_HID), lambda i: (0, 0)),
            pl.BlockSpec((NC, BLK, 16), lambda i: (0, i, 0)),
        ],
        out_specs=pl.BlockSpec((BLK, D_HID), lambda i: (i, 0)),
        out_shape=jax.ShapeDtypeStruct((N_NODES, D_HID), jnp.float32),
    )(feature, W1, degp)


def _mid_body(z1_ref, p_ref, degp_ref, b1_ref, w2_ref, o_ref):
    dis = _dis_from(degp_ref)
    accv = z1_ref[...] + p_ref[0] + p_ref[1]
    h = jnp.maximum(accv * dis + b1_ref[...], 0.0)
    y2 = jnp.dot(h, w2_ref[...], preferred_element_type=jnp.float32)
    o_ref[...] = y2 * dis


def _tc_mid(z1, p, degp, b1, W2):
    return pl.pallas_call(
        _mid_body,
        grid=GRID,
        in_specs=[
            pl.BlockSpec((BLK, D_HID), lambda i: (i, 0)),
            pl.BlockSpec((NC, BLK, D_HID), lambda i: (0, i, 0)),
            pl.BlockSpec((NC, BLK, 16), lambda i: (0, i, 0)),
            pl.BlockSpec((1, D_HID), lambda i: (0, 0)),
            pl.BlockSpec((D_HID, D_OUT), lambda i: (0, 0)),
        ],
        out_specs=pl.BlockSpec((BLK, D_OUT), lambda i: (i, 0)),
        out_shape=jax.ShapeDtypeStruct((N_NODES, D_OUT), jnp.float32),
    )(z1, p, degp, b1, W2)


def _final_body(z2_ref, q_ref, degp_ref, b2_ref, o_ref):
    dis = _dis_from(degp_ref)
    accv = z2_ref[...] + q_ref[0] + q_ref[1]
    o_ref[...] = accv * dis + b2_ref[...]


def _tc_final(z2, q, degp, b2):
    return pl.pallas_call(
        _final_body,
        grid=GRID,
        in_specs=[
            pl.BlockSpec((BLK, D_OUT), lambda i: (i, 0)),
            pl.BlockSpec((NC, BLK, D_OUT), lambda i: (0, i, 0)),
            pl.BlockSpec((NC, BLK, 16), lambda i: (0, i, 0)),
            pl.BlockSpec((1, D_OUT), lambda i: (0, 0)),
        ],
        out_specs=pl.BlockSpec((BLK, D_OUT), lambda i: (i, 0)),
        out_shape=jax.ShapeDtypeStruct((N_NODES, D_OUT), jnp.float32),
    )(z2, q, degp, b2)


def kernel(edge, feature, W1, b1, W2, b2):
    edge = edge.astype(jnp.int32)
    src1 = edge[0]
    dst1 = edge[1]
    zeros128 = jnp.zeros((N_NODES, D_HID), jnp.float32)
    zeros16 = jnp.zeros((N_NODES, 16), jnp.float32)

    degp = _sc_degree(dst1, zeros16)
    z1 = _tc_lin1(feature, W1, degp)
    p = _sc_conv128(src1, dst1, z1, zeros128)
    z2 = _tc_mid(z1, p, degp, b1.reshape(1, -1), W2)
    q = _sc_conv16(src1, dst1, z2, zeros16)
    return _tc_final(z2, q, degp, b2.reshape(1, -1))


# speedup vs baseline: 38.4231x; 1.6766x over previous
import functools

import jax
import jax.numpy as jnp
from jax import lax
from jax.experimental import pallas as pl
from jax.experimental.pallas import tpu as pltpu
from jax.experimental.pallas import tpu_sc as plsc

N_NODES = 10000
N_EDGES = 320000
D_IN = 128
D_HID = 128
D_OUT = 16

NC = 2
NS = 16
NW = NC * NS
E_PER_TILE = N_EDGES // NW
K = 80
G = E_PER_TILE // K
ROWS_PER_TILE = N_NODES // NS
NBUF = 5
NSG = G // NBUF

_mesh = plsc.VectorSubcoreMesh(core_axis_name="c", subcore_axis_name="s")


def _sc_degree(dst_hbm, zeros_hbm, out_hbm, dst_v, ones_v, acc, ssem):
    c = lax.axis_index("c")
    s = lax.axis_index("s")
    wid = s * NC + c
    pltpu.sync_copy(dst_hbm.at[pl.ds(wid * E_PER_TILE, E_PER_TILE)], dst_v)

    def fill(i, carry):
        ones_v[i, :] = jnp.full((16,), 1.0, jnp.float32)
        return carry

    lax.fori_loop(0, K, fill, 0)
    pltpu.sync_copy(
        zeros_hbm.at[pl.ds(s * ROWS_PER_TILE, ROWS_PER_TILE)],
        acc.at[pl.ds(s * ROWS_PER_TILE, ROWS_PER_TILE)],
    )
    plsc.subcore_barrier()

    def fire(j):
        for b in range(NBUF):
            g = j * NBUF + b
            pltpu.async_copy(
                ones_v, acc.at[dst_v.at[pl.ds(g * K, K)]], ssem, add=True)

    def drain():
        for b in range(NBUF):
            pltpu.make_async_copy(
                zeros_hbm.at[pl.ds(0, K)], ones_v, ssem).wait()

    fire(0)

    def body(j, carry):
        drain()
        fire(j)
        return carry

    lax.fori_loop(1, NSG, body, 0)
    drain()
    plsc.subcore_barrier()
    pltpu.sync_copy(
        acc.at[pl.ds(s * ROWS_PER_TILE, ROWS_PER_TILE)],
        out_hbm.at[c, pl.ds(s * ROWS_PER_TILE, ROWS_PER_TILE)],
    )


_sc_degree = functools.partial(
    pl.kernel,
    out_type=jax.ShapeDtypeStruct((NC, N_NODES, 16), jnp.float32),
    mesh=_mesh,
    compiler_params=pltpu.CompilerParams(use_tc_tiling_on_sc=False),
    scratch_types=[
        pltpu.VMEM((E_PER_TILE,), jnp.int32),
        pltpu.VMEM((K, 16), jnp.float32),
        pltpu.VMEM_SHARED((N_NODES, 16), jnp.float32),
        pltpu.SemaphoreType.DMA,
    ],
)(_sc_degree)


def _make_sc_conv(width, k, nbuf):
    @functools.partial(
        pl.kernel,
        out_type=jax.ShapeDtypeStruct((NC, N_NODES, width), jnp.float32),
        mesh=_mesh,
        compiler_params=pltpu.CompilerParams(use_tc_tiling_on_sc=False),
        scratch_types=[
            pltpu.VMEM((E_PER_TILE,), jnp.int32),
            pltpu.VMEM((E_PER_TILE,), jnp.int32),
            pltpu.VMEM((nbuf, k, width), jnp.float32),
            pltpu.VMEM_SHARED((N_NODES, width), jnp.float32),
            pltpu.SemaphoreType.DMA((nbuf,)),
            pltpu.SemaphoreType.DMA,
        ],
    )
    def conv(src_hbm, dst_hbm, z_hbm, zeros_hbm, out_hbm,
             src_v, dst_v, gbufs, acc, gsem, ssem):
        c = lax.axis_index("c")
        s = lax.axis_index("s")
        wid = s * NC + c
        pltpu.sync_copy(src_hbm.at[pl.ds(wid * E_PER_TILE, E_PER_TILE)], src_v)
        pltpu.sync_copy(dst_hbm.at[pl.ds(wid * E_PER_TILE, E_PER_TILE)], dst_v)
        pltpu.sync_copy(
            zeros_hbm.at[pl.ds(s * ROWS_PER_TILE, ROWS_PER_TILE)],
            acc.at[pl.ds(s * ROWS_PER_TILE, ROWS_PER_TILE)],
        )
        plsc.subcore_barrier()

        def round_(j, first):
            descs = []
            for b in range(nbuf):
                if not first:
                    pltpu.make_async_copy(
                        z_hbm.at[pl.ds(0, k)], gbufs.at[b], ssem).wait()
                g = j * nbuf + b
                descs.append(pltpu.async_copy(
                    z_hbm.at[src_v.at[pl.ds(g * k, k)]],
                    gbufs.at[b], gsem.at[b]))
            for b in range(nbuf):
                descs[b].wait()
                g = j * nbuf + b
                pltpu.async_copy(
                    gbufs.at[b], acc.at[dst_v.at[pl.ds(g * k, k)]],
                    ssem, add=True)

        round_(0, True)

        def body(j, carry):
            round_(j, False)
            return carry

        nsg = E_PER_TILE // k // nbuf
        lax.fori_loop(1, nsg, body, 0)
        for b in range(nbuf):
            pltpu.make_async_copy(
                z_hbm.at[pl.ds(0, k)], gbufs.at[b], ssem).wait()
        plsc.subcore_barrier()
        pltpu.sync_copy(
            acc.at[pl.ds(s * ROWS_PER_TILE, ROWS_PER_TILE)],
            out_hbm.at[c, pl.ds(s * ROWS_PER_TILE, ROWS_PER_TILE)],
        )

    return conv


_sc_conv128 = _make_sc_conv(D_HID, 40, 5)
_sc_conv16 = _make_sc_conv(D_OUT, 80, 5)


BLK = 1000
GRID = (N_NODES // BLK,)


def _dis_from(degp_ref):
    deg = degp_ref[0, :, 0:1] + degp_ref[1, :, 0:1] + 1.0
    return lax.rsqrt(deg)


def _lin1_body(x_ref, w_ref, degp_ref, o_ref):
    y = jnp.dot(x_ref[...], w_ref[...], preferred_element_type=jnp.float32)
    o_ref[...] = y * _dis_from(degp_ref)


def _tc_lin1(feature, W1, degp):
    return pl.pallas_call(
        _lin1_body,
        grid=GRID,
        in_specs=[
            pl.BlockSpec((BLK, D_IN), lambda i: (i, 0)),
            pl.BlockSpec((D_IN, D_HID), lambda i: (0, 0)),
            pl.BlockSpec((NC, BLK, 16), lambda i: (0, i, 0)),
        ],
        out_specs=pl.BlockSpec((BLK, D_HID), lambda i: (i, 0)),
        out_shape=jax.ShapeDtypeStruct((N_NODES, D_HID), jnp.float32),
    )(feature, W1, degp)


def _mid_body(z1_ref, p_ref, degp_ref, b1_ref, w2_ref, o_ref):
    dis = _dis_from(degp_ref)
    accv = z1_ref[...] + p_ref[0] + p_ref[1]
    h = jnp.maximum(accv * dis + b1_ref[...], 0.0)
    y2 = jnp.dot(h, w2_ref[...], preferred_element_type=jnp.float32)
    o_ref[...] = y2 * dis


def _tc_mid(z1, p, degp, b1, W2):
    return pl.pallas_call(
        _mid_body,
        grid=GRID,
        in_specs=[
            pl.BlockSpec((BLK, D_HID), lambda i: (i, 0)),
            pl.BlockSpec((NC, BLK, D_HID), lambda i: (0, i, 0)),
            pl.BlockSpec((NC, BLK, 16), lambda i: (0, i, 0)),
            pl.BlockSpec((1, D_HID), lambda i: (0, 0)),
            pl.BlockSpec((D_HID, D_OUT), lambda i: (0, 0)),
        ],
        out_specs=pl.BlockSpec((BLK, D_OUT), lambda i: (i, 0)),
        out_shape=jax.ShapeDtypeStruct((N_NODES, D_OUT), jnp.float32),
    )(z1, p, degp, b1, W2)


def _final_body(z2_ref, q_ref, degp_ref, b2_ref, o_ref):
    dis = _dis_from(degp_ref)
    accv = z2_ref[...] + q_ref[0] + q_ref[1]
    o_ref[...] = accv * dis + b2_ref[...]


def _tc_final(z2, q, degp, b2):
    return pl.pallas_call(
        _final_body,
        grid=GRID,
        in_specs=[
            pl.BlockSpec((BLK, D_OUT), lambda i: (i, 0)),
            pl.BlockSpec((NC, BLK, D_OUT), lambda i: (0, i, 0)),
            pl.BlockSpec((NC, BLK, 16), lambda i: (0, i, 0)),
            pl.BlockSpec((1, D_OUT), lambda i: (0, 0)),
        ],
        out_specs=pl.BlockSpec((BLK, D_OUT), lambda i: (i, 0)),
        out_shape=jax.ShapeDtypeStruct((N_NODES, D_OUT), jnp.float32),
    )(z2, q, degp, b2)


def kernel(edge, feature, W1, b1, W2, b2):
    edge = edge.astype(jnp.int32)
    src1 = edge[0]
    dst1 = edge[1]
    zeros128 = jnp.zeros((N_NODES, D_HID), jnp.float32)
    zeros16 = jnp.zeros((N_NODES, 16), jnp.float32)

    degp = _sc_degree(dst1, zeros16)
    z1 = _tc_lin1(feature, W1, degp)
    p = _sc_conv128(src1, dst1, z1, zeros128)
    z2 = _tc_mid(z1, p, degp, b1.reshape(1, -1), W2)
    q = _sc_conv16(src1, dst1, z2, zeros16)
    return _tc_final(z2, q, degp, b2.reshape(1, -1))
